# baseline (device time: 131393 ns/iter reference)
import jax
import jax.numpy as jnp
from jax import lax
from jax.experimental import pallas as pl
from jax.experimental.pallas import tpu as pltpu

N_DEV = 4
M_PER = 1024
D = 1024
F = 4096
TILE = 256
NT = M_PER // TILE


def kernel(x, W1, W2):
    xb = x.astype(jnp.bfloat16)

    def body(x_ref, w1_hbm, w2_hbm, out_ref, xg, prec, w1b, w2b,
             stw1, stw2, ag_ssem, ag_rsem, rs_ssem, rs_rsem, w_sem):
        i = lax.axis_index("i")

        barrier = pltpu.get_barrier_semaphore()
        for d in (1, 2, 3):
            pl.semaphore_signal(
                barrier, inc=1,
                device_id=((i + d) % N_DEV,),
                device_id_type=pl.DeviceIdType.MESH,
            )
        pl.semaphore_wait(barrier, 3)

        def ag_copy(d, t):
            return pltpu.make_async_remote_copy(
                src_ref=x_ref.at[pl.ds(t * TILE, TILE), :],
                dst_ref=xg.at[3 - d, pl.ds(t * TILE, TILE), :],
                send_sem=ag_ssem.at[d - 1, t],
                recv_sem=ag_rsem.at[3 - d, t],
                device_id=((i + d) % N_DEV,),
                device_id_type=pl.DeviceIdType.MESH,
            )

        def rs_copy(s, t):
            return pltpu.make_async_remote_copy(
                src_ref=xg.at[s, pl.ds(t * TILE, TILE), :],
                dst_ref=prec.at[s, pl.ds(t * TILE, TILE), :],
                send_sem=rs_ssem.at[s, t],
                recv_sem=rs_rsem.at[s, t],
                device_id=((i + s + 1) % N_DEV,),
                device_id_type=pl.DeviceIdType.MESH,
            )

        for t in range(NT):
            ag_copy(2, t).start()

        def stream(src, stage, dst, rows, ntiles):
            def mk(t):
                return pltpu.make_async_copy(
                    src.at[pl.ds(t * rows, rows), :],
                    stage.at[t % 2],
                    w_sem.at[t % 2],
                )
            mk(0).start()

            def step(t, carry):
                mk(t).start()
                mk(t - 1).wait()
                dst[pl.ds((t - 1) * rows, rows), :] = (
                    stage[(t - 1) % 2].astype(jnp.bfloat16))
                return carry
            lax.fori_loop(1, ntiles, step, 0)
            mk(ntiles - 1).wait()
            dst[pl.ds((ntiles - 1) * rows, rows), :] = (
                stage[(ntiles - 1) % 2].astype(jnp.bfloat16))

        stream(w1_hbm, stw1, w1b, 128, 8)

        for t in range(NT):
            ag_copy(1, t).start()
            ag_copy(3, t).start()

        stream(w2_hbm, stw2, w2b, 256, 16)

        FSLAB = 1024

        def gemm(xs):
            def fstep(f, acc):
                f0 = f * FSLAB
                h = jnp.dot(xs, w1b[:, pl.ds(f0, FSLAB)],
                            preferred_element_type=jnp.float32)
                h = h * jax.nn.sigmoid(h)
                return acc + jnp.dot(h.astype(jnp.bfloat16),
                                     w2b[pl.ds(f0, FSLAB), :],
                                     preferred_element_type=jnp.float32)
            return lax.fori_loop(0, F // FSLAB, fstep,
                                 jnp.zeros((TILE, D), jnp.float32))

        for s, d in ((1, 2), (0, 3), (2, 1)):
            def step(t, carry, s=s, d=d):
                t0 = t * TILE
                ag_copy(d, t).wait_recv()
                c = gemm(xg[s, pl.ds(t0, TILE), :])
                xg[s, pl.ds(t0, TILE), :] = c.astype(jnp.bfloat16)
                rs_copy(s, t).start()
                return carry
            lax.fori_loop(0, NT, step, 0)

        def own_step(t, carry):
            t0 = t * TILE
            out_ref[pl.ds(t0, TILE), :] = gemm(x_ref[pl.ds(t0, TILE), :])
            return carry
        lax.fori_loop(0, NT, own_step, 0)

        for s in range(3):
            def wrecv(t, carry, s=s):
                rs_copy(s, t).wait_recv()
                return carry
            lax.fori_loop(0, NT, wrecv, 0)
        out_ref[...] = (out_ref[...]
                        + prec[0].astype(jnp.float32)
                        + prec[1].astype(jnp.float32)
                        + prec[2].astype(jnp.float32))

        for d in (1, 2, 3):
            def wsend_ag(t, carry, d=d):
                ag_copy(d, t).wait_send()
                return carry
            lax.fori_loop(0, NT, wsend_ag, 0)
        for s in range(3):
            def wsend_rs(t, carry, s=s):
                rs_copy(s, t).wait_send()
                return carry
            lax.fori_loop(0, NT, wsend_rs, 0)

    return pl.pallas_call(
        body,
        out_shape=jax.ShapeDtypeStruct((M_PER, D), jnp.float32),
        in_specs=[
            pl.BlockSpec(memory_space=pltpu.VMEM),
            pl.BlockSpec(memory_space=pl.ANY),
            pl.BlockSpec(memory_space=pl.ANY),
        ],
        out_specs=pl.BlockSpec(memory_space=pltpu.VMEM),
        scratch_shapes=[
            pltpu.VMEM((3, M_PER, D), jnp.bfloat16),
            pltpu.VMEM((3, M_PER, D), jnp.bfloat16),
            pltpu.VMEM((D, F), jnp.bfloat16),
            pltpu.VMEM((F, D), jnp.bfloat16),
            pltpu.VMEM((2, 128, F), jnp.float32),
            pltpu.VMEM((2, 256, D), jnp.float32),
            pltpu.SemaphoreType.DMA((3, NT)),
            pltpu.SemaphoreType.DMA((3, NT)),
            pltpu.SemaphoreType.DMA((3, NT)),
            pltpu.SemaphoreType.DMA((3, NT)),
            pltpu.SemaphoreType.DMA((2,)),
        ],
        compiler_params=pltpu.CompilerParams(
            collective_id=0,
            vmem_limit_bytes=56 * 1024 * 1024,
        ),
    )(xb, W1, W2)


# device time: 108107 ns/iter; 1.2154x vs baseline; 1.2154x over previous
import jax
import jax.numpy as jnp
from jax import lax
from jax.experimental import pallas as pl
from jax.experimental.pallas import tpu as pltpu

N_DEV = 4
M_PER = 1024
D = 1024
F = 4096
TILE = 256
NT = M_PER // TILE


def kernel(x, W1, W2):
    xb = x.astype(jnp.bfloat16)

    def body(x_ref, w1_hbm, w2_hbm, out_ref, xg, prec, w1b, w2b,
             stw1, stw2, ag_ssem, ag_rsem, rs_ssem, rs_rsem, w_sem):
        i = lax.axis_index("i")

        barrier = pltpu.get_barrier_semaphore()
        for d in (1, 2, 3):
            pl.semaphore_signal(
                barrier, inc=1,
                device_id=((i + d) % N_DEV,),
                device_id_type=pl.DeviceIdType.MESH,
            )
        pl.semaphore_wait(barrier, 3)

        def ag_copy(d, t):
            return pltpu.make_async_remote_copy(
                src_ref=x_ref.at[pl.ds(t * TILE, TILE), :],
                dst_ref=xg.at[3 - d, pl.ds(t * TILE, TILE), :],
                send_sem=ag_ssem.at[d - 1, t],
                recv_sem=ag_rsem.at[3 - d, t],
                device_id=((i + d) % N_DEV,),
                device_id_type=pl.DeviceIdType.MESH,
            )

        def rs_copy(s, t):
            return pltpu.make_async_remote_copy(
                src_ref=xg.at[s, pl.ds(t * TILE, TILE), :],
                dst_ref=prec.at[s, pl.ds(t * TILE, TILE), :],
                send_sem=rs_ssem.at[s, t],
                recv_sem=rs_rsem.at[s, t],
                device_id=((i + s + 1) % N_DEV,),
                device_id_type=pl.DeviceIdType.MESH,
            )

        for t in range(NT):
            ag_copy(2, t).start()

        def stream(src, stage, dst, rows, ntiles):
            def mk(t):
                return pltpu.make_async_copy(
                    src.at[pl.ds(t * rows, rows), :],
                    stage.at[t % 2],
                    w_sem.at[t % 2],
                )
            mk(0).start()

            def step(t, carry):
                mk(t).start()
                mk(t - 1).wait()
                dst[pl.ds((t - 1) * rows, rows), :] = (
                    stage[(t - 1) % 2].astype(jnp.bfloat16))
                return carry
            lax.fori_loop(1, ntiles, step, 0)
            mk(ntiles - 1).wait()
            dst[pl.ds((ntiles - 1) * rows, rows), :] = (
                stage[(ntiles - 1) % 2].astype(jnp.bfloat16))

        stream(w1_hbm, stw1, w1b, 128, 8)

        for t in range(NT):
            ag_copy(1, t).start()
            ag_copy(3, t).start()

        stream(w2_hbm, stw2, w2b, 256, 16)

        def gemm(xs):
            h = jnp.dot(xs, w1b[...], preferred_element_type=jnp.float32)
            h = h * jax.nn.sigmoid(h)
            return jnp.dot(h.astype(jnp.bfloat16), w2b[...],
                           preferred_element_type=jnp.float32)

        for s, d in ((1, 2), (0, 3), (2, 1)):
            def step(t, carry, s=s, d=d):
                t0 = t * TILE
                ag_copy(d, t).wait_recv()
                c = gemm(xg[s, pl.ds(t0, TILE), :])
                xg[s, pl.ds(t0, TILE), :] = c.astype(jnp.bfloat16)
                rs_copy(s, t).start()
                return carry
            lax.fori_loop(0, NT, step, 0)

        def own_step(t, carry):
            t0 = t * TILE
            out_ref[pl.ds(t0, TILE), :] = gemm(x_ref[pl.ds(t0, TILE), :])
            return carry
        lax.fori_loop(0, NT, own_step, 0)

        for s in range(3):
            def wrecv(t, carry, s=s):
                rs_copy(s, t).wait_recv()
                return carry
            lax.fori_loop(0, NT, wrecv, 0)
        out_ref[...] = (out_ref[...]
                        + prec[0].astype(jnp.float32)
                        + prec[1].astype(jnp.float32)
                        + prec[2].astype(jnp.float32))

        for d in (1, 2, 3):
            def wsend_ag(t, carry, d=d):
                ag_copy(d, t).wait_send()
                return carry
            lax.fori_loop(0, NT, wsend_ag, 0)
        for s in range(3):
            def wsend_rs(t, carry, s=s):
                rs_copy(s, t).wait_send()
                return carry
            lax.fori_loop(0, NT, wsend_rs, 0)

    return pl.pallas_call(
        body,
        out_shape=jax.ShapeDtypeStruct((M_PER, D), jnp.float32),
        in_specs=[
            pl.BlockSpec(memory_space=pltpu.VMEM),
            pl.BlockSpec(memory_space=pl.ANY),
            pl.BlockSpec(memory_space=pl.ANY),
        ],
        out_specs=pl.BlockSpec(memory_space=pltpu.VMEM),
        scratch_shapes=[
            pltpu.VMEM((3, M_PER, D), jnp.bfloat16),
            pltpu.VMEM((3, M_PER, D), jnp.bfloat16),
            pltpu.VMEM((D, F), jnp.bfloat16),
            pltpu.VMEM((F, D), jnp.bfloat16),
            pltpu.VMEM((2, 128, F), jnp.float32),
            pltpu.VMEM((2, 256, D), jnp.float32),
            pltpu.SemaphoreType.DMA((3, NT)),
            pltpu.SemaphoreType.DMA((3, NT)),
            pltpu.SemaphoreType.DMA((3, NT)),
            pltpu.SemaphoreType.DMA((3, NT)),
            pltpu.SemaphoreType.DMA((2,)),
        ],
        compiler_params=pltpu.CompilerParams(
            collective_id=0,
            vmem_limit_bytes=56 * 1024 * 1024,
        ),
    )(xb, W1, W2)
